# baseline (speedup 1.0000x reference)
import functools
import jax
import jax.numpy as jnp
from jax import lax
from jax.experimental import pallas as pl
from jax.experimental.pallas import tpu as pltpu
from jax.experimental.pallas import tpu_sc as plsc

_K = 27
_COUT = 16
_EPS = 1e-5
_NC = 2
_NS = 16
_NW = _NC * _NS
_CHUNK = 128
_ZC = 64

_N0, _N1, _N2, _N3 = 100000, 25000, 6250, 1563
_NP1, _NP2, _NP3 = 25600, 7168, 2048
_E1P, _E2P, _E3P = 679936, 172032, 49152


def _make_edge_conv(e_pad, np_out):
    e_per_w = e_pad // _NW
    nch = e_per_w // _CHUNK
    npairs = nch // 2
    rpt = np_out // _NS
    zc = next(d for d in (320, 256, 224, 128, 64, 32, 16) if rpt % d == 0)
    mesh = plsc.VectorSubcoreMesh(core_axis_name="c", subcore_axis_name="s")

    @functools.partial(
        pl.kernel,
        out_type=jax.ShapeDtypeStruct((_NC, np_out, _COUT), jnp.float32),
        mesh=mesh,
        compiler_params=pltpu.CompilerParams(use_tc_tiling_on_sc=False),
        scratch_types=[
            pltpu.VMEM_SHARED((np_out, _COUT), jnp.float32),
            pltpu.VMEM((3, _CHUNK), jnp.int32),
            pltpu.VMEM((3, _CHUNK), jnp.int32),
            pltpu.VMEM((_CHUNK,), jnp.int32),
            pltpu.VMEM((_CHUNK,), jnp.int32),
            pltpu.VMEM((_CHUNK,), jnp.int32),
            pltpu.VMEM((_CHUNK,), jnp.int32),
            pltpu.VMEM((_CHUNK, _COUT), jnp.float32),
            pltpu.VMEM((_CHUNK, _COUT), jnp.float32),
            pltpu.VMEM((zc, _COUT), jnp.float32),
            pltpu.SemaphoreType.DMA,
            pltpu.SemaphoreType.DMA,
            pltpu.SemaphoreType.DMA,
            pltpu.SemaphoreType.DMA,
            pltpu.SemaphoreType.DMA,
            pltpu.SemaphoreType.DMA,
        ],
    )
    def conv(tab_hbm, eidx_hbm, out_hbm,
             acc, ibuf0, ibuf1, gbuf0, gbuf1, dbuf0, dbuf1, rows0, rows1,
             zero_v, isem0, isem1, gsem0, gsem1, ssem0, ssem1):
        cid = lax.axis_index("c")
        sid = lax.axis_index("s")
        wid = sid * _NC + cid

        ibuf = (ibuf0, ibuf1)
        gbuf = (gbuf0, gbuf1)
        dbuf = (dbuf0, dbuf1)
        rows = (rows0, rows1)
        isem = (isem0, isem1)
        gsem = (gsem0, gsem1)
        ssem = (ssem0, ssem1)

        def zfill(r, carry):
            zero_v[r] = jnp.zeros((_COUT,), jnp.float32)
            return carry
        lax.fori_loop(0, zc, zfill, 0)

        for i in range(rpt // zc):
            pltpu.async_copy(zero_v, acc.at[pl.ds(sid * rpt + i * zc, zc)],
                             gsem0)
        for i in range(rpt // zc):
            pltpu.make_async_copy(zero_v, acc.at[pl.ds(sid * rpt, zc)],
                                  gsem0).wait()
        plsc.subcore_barrier()

        chbase = wid * nch

        def fire_idx(g, b):
            return pltpu.async_copy(eidx_hbm.at[chbase + g], ibuf[b], isem[b])

        def wait_idx(b):
            pltpu.make_async_copy(eidx_hbm.at[chbase], ibuf[b], isem[b]).wait()

        def fire_gat(b):
            return pltpu.async_copy(tab_hbm.at[gbuf[b]], rows[b], gsem[b])

        def wait_gat(b):
            pltpu.make_async_copy(tab_hbm.at[gbuf[b]], rows[b], gsem[b]).wait()

        def fire_sct(b):
            return pltpu.async_copy(rows[b], acc.at[dbuf[b]], ssem[b], add=True)

        def wait_sct(b):
            pltpu.make_async_copy(rows[b], acc.at[dbuf[b]], ssem[b]).wait()

        def compute_idx(b):
            for j in range(_CHUNK // 16):
                sl = pl.ds(j * 16, 16)
                s = ibuf[b][0, sl]
                k = ibuf[b][1, sl]
                gbuf[b][sl] = s * _K + k
                dbuf[b][sl] = ibuf[b][2, sl]

        fire_idx(0, 0)

        def pair(p, carry):
            for b in (0, 1):
                g = 2 * p + b
                wait_idx(b)

                @pl.when(p >= 1)
                def _():
                    wait_sct(b)

                compute_idx(b)
                fire_gat(b)
                if b == 0:
                    fire_idx(g + 1, 1)
                else:
                    @pl.when(p < npairs - 1)
                    def _():
                        fire_idx(g + 1, 0)

                if b == 0:
                    @pl.when(p >= 1)
                    def _():
                        wait_gat(1)
                        fire_sct(1)
                else:
                    wait_gat(0)
                    fire_sct(0)
            return carry

        lax.fori_loop(0, npairs, pair, 0)

        wait_gat(1)
        fire_sct(1)
        wait_sct(0)
        wait_sct(1)

        plsc.subcore_barrier()

        for i in range(rpt // zc):
            base = sid * rpt + i * zc
            pltpu.async_copy(acc.at[pl.ds(base, zc)],
                             out_hbm.at[cid, pl.ds(base, zc)], gsem0)
        for i in range(rpt // zc):
            base = sid * rpt
            pltpu.make_async_copy(acc.at[pl.ds(base, zc)],
                                  out_hbm.at[cid, pl.ds(base, zc)],
                                  gsem0).wait()

    return conv


def _make_edge_conv_mv(e_pad, np_out, cin, unroll):
    e_per_w = e_pad // _NW
    nch = e_per_w // _CHUNK
    npairs = nch // 2
    rpt = np_out // _NS
    zc = next(d for d in (320, 256, 224, 128, 64, 32, 16) if rpt % d == 0)
    mesh = plsc.VectorSubcoreMesh(core_axis_name="c", subcore_axis_name="s")

    @functools.partial(
        pl.kernel,
        out_type=jax.ShapeDtypeStruct((_NC, np_out, _COUT), jnp.float32),
        mesh=mesh,
        compiler_params=pltpu.CompilerParams(use_tc_tiling_on_sc=False),
        scratch_types=[
            pltpu.VMEM_SHARED((np_out, _COUT), jnp.float32),
            pltpu.VMEM((3, _CHUNK), jnp.int32),
            pltpu.VMEM((3, _CHUNK), jnp.int32),
            pltpu.VMEM((_CHUNK,), jnp.int32),
            pltpu.VMEM((_CHUNK,), jnp.int32),
            pltpu.VMEM((_CHUNK,), jnp.int32),
            pltpu.VMEM((_CHUNK,), jnp.int32),
            pltpu.VMEM((_CHUNK,), jnp.int32),
            pltpu.VMEM((_CHUNK,), jnp.int32),
            pltpu.VMEM((_CHUNK, _COUT), jnp.float32),
            pltpu.VMEM((_CHUNK, _COUT), jnp.float32),
            pltpu.VMEM((_CHUNK, _COUT), jnp.float32),
            pltpu.VMEM((_CHUNK, _COUT), jnp.float32),
            pltpu.VMEM((_K * cin, _COUT), jnp.float32),
            pltpu.VMEM((zc, _COUT), jnp.float32),
            pltpu.SemaphoreType.DMA,
            pltpu.SemaphoreType.DMA,
            pltpu.SemaphoreType.DMA,
            pltpu.SemaphoreType.DMA,
            pltpu.SemaphoreType.DMA,
            pltpu.SemaphoreType.DMA,
        ],
    )
    def conv1(tab_hbm, eidx_hbm, w_hbm, out_hbm,
              acc, ibuf0, ibuf1, sbuf0, sbuf1, kbuf0, kbuf1, dbuf0, dbuf1,
              xrows0, xrows1, msgs0, msgs1, w_v, zero_v,
              isem0, isem1, gsem0, gsem1, ssem0, ssem1):
        cid = lax.axis_index("c")
        sid = lax.axis_index("s")
        wid = sid * _NC + cid

        ibuf = (ibuf0, ibuf1)
        sbuf = (sbuf0, sbuf1)
        kbuf = (kbuf0, kbuf1)
        dbuf = (dbuf0, dbuf1)
        xrows = (xrows0, xrows1)
        msgs = (msgs0, msgs1)
        isem = (isem0, isem1)
        gsem = (gsem0, gsem1)
        ssem = (ssem0, ssem1)

        pltpu.sync_copy(w_hbm, w_v)

        def zfill(r, carry):
            zero_v[r] = jnp.zeros((_COUT,), jnp.float32)
            return carry
        lax.fori_loop(0, zc, zfill, 0)

        for i in range(rpt // zc):
            pltpu.async_copy(zero_v, acc.at[pl.ds(sid * rpt + i * zc, zc)],
                             gsem0)
        for i in range(rpt // zc):
            pltpu.make_async_copy(zero_v, acc.at[pl.ds(sid * rpt, zc)],
                                  gsem0).wait()
        plsc.subcore_barrier()

        chbase = wid * nch

        def fire_idx(g, b):
            return pltpu.async_copy(eidx_hbm.at[chbase + g], ibuf[b], isem[b])

        def wait_idx(b):
            pltpu.make_async_copy(eidx_hbm.at[chbase], ibuf[b], isem[b]).wait()

        def fire_gat(b):
            return pltpu.async_copy(tab_hbm.at[sbuf[b]], xrows[b], gsem[b])

        def wait_gat(b):
            pltpu.make_async_copy(tab_hbm.at[sbuf[b]], xrows[b], gsem[b]).wait()

        def fire_sct(b):
            return pltpu.async_copy(msgs[b], acc.at[dbuf[b]], ssem[b], add=True)

        def wait_sct(b):
            pltpu.make_async_copy(msgs[b], acc.at[dbuf[b]], ssem[b]).wait()

        def stage_idx(b):
            for j in range(_CHUNK // 16):
                sl = pl.ds(j * 16, 16)
                sbuf[b][sl] = ibuf[b][0, sl]
                kbuf[b][sl] = ibuf[b][1, sl]
                dbuf[b][sl] = ibuf[b][2, sl]

        def compute_msgs(b):
            @plsc.parallel_loop(0, _CHUNK // 16, unroll=unroll)
            def grp(j):
                kvec = kbuf[b][pl.ds(j * 16, 16)] * cin
                for l in range(16):
                    wb = kvec[l]
                    xrow = xrows[b][j * 16 + l]
                    m = xrow[0] * w_v[wb]
                    for i in range(1, cin):
                        m = m + xrow[i] * w_v[wb + i]
                    msgs[b][j * 16 + l] = m

        fire_idx(0, 0)

        def pair(p, carry):
            for b in (0, 1):
                g = 2 * p + b
                wait_idx(b)

                @pl.when(p >= 1)
                def _():
                    wait_sct(b)

                stage_idx(b)
                fire_gat(b)
                if b == 0:
                    fire_idx(g + 1, 1)

                    @pl.when(p >= 1)
                    def _():
                        wait_gat(1)
                        compute_msgs(1)
                        fire_sct(1)
                else:
                    @pl.when(p < npairs - 1)
                    def _():
                        fire_idx(g + 1, 0)
                    wait_gat(0)
                    compute_msgs(0)
                    fire_sct(0)
            return carry

        lax.fori_loop(0, npairs, pair, 0)

        wait_gat(1)
        compute_msgs(1)
        fire_sct(1)
        wait_sct(0)
        wait_sct(1)

        plsc.subcore_barrier()

        for i in range(rpt // zc):
            base = sid * rpt + i * zc
            pltpu.async_copy(acc.at[pl.ds(base, zc)],
                             out_hbm.at[cid, pl.ds(base, zc)], gsem0)
        for i in range(rpt // zc):
            base = sid * rpt
            pltpu.make_async_copy(acc.at[pl.ds(base, zc)],
                                  out_hbm.at[cid, pl.ds(base, zc)],
                                  gsem0).wait()

    return conv1


def _expand_x(x, wflat):
    bm, grid = 2000, _N0 // 2000

    def body(x_ref, w_ref, y_ref):
        y_ref[...] = jnp.dot(x_ref[...], w_ref[...],
                             preferred_element_type=jnp.float32)

    return pl.pallas_call(
        body,
        grid=(grid,),
        in_specs=[pl.BlockSpec((bm, 3), lambda i: (i, 0)),
                  pl.BlockSpec((3, _K * _COUT), lambda i: (0, 0))],
        out_specs=pl.BlockSpec((bm, _K * _COUT), lambda i: (i, 0)),
        out_shape=jax.ShapeDtypeStruct((_N0, _K * _COUT), jnp.float32),
    )(x, wflat)


def _norm_only(hp, alpha, n_valid):
    npad = hp.shape[1]

    def body(hp_ref, a_ref, o_ref):
        h = hp_ref[0] + hp_ref[1]
        rows = lax.broadcasted_iota(jnp.int32, (npad, 1), 0)
        m = (rows < n_valid).astype(jnp.float32)
        hm = h * m
        s1 = jnp.sum(hm, axis=0, keepdims=True)
        s2 = jnp.sum(hm * hm, axis=0, keepdims=True)
        mu = s1 / n_valid
        var = s2 / n_valid - mu * mu
        inv = lax.rsqrt(var + _EPS)
        hn = (h - mu) * inv
        a = a_ref[0]
        o_ref[...] = jnp.where(hn > 0, hn, a * hn)

    return pl.pallas_call(
        body,
        grid=(1,),
        in_specs=[
            pl.BlockSpec((2, npad, _COUT), lambda i: (0, 0, 0)),
            pl.BlockSpec(memory_space=pltpu.SMEM),
        ],
        out_specs=pl.BlockSpec((npad, _COUT), lambda i: (0, 0)),
        out_shape=jax.ShapeDtypeStruct((npad, _COUT), jnp.float32),
    )(hp, alpha)


def _norm_expand(hp, wf, alpha, n_valid, bm):
    npad = hp.shape[1]
    cout = wf.shape[1]
    grid = npad // bm

    def body(hp_ref, w_ref, a_ref, o_ref, st_ref):
        i = pl.program_id(0)

        @pl.when(i == 0)
        def _():
            h = hp_ref[0] + hp_ref[1]
            rows = lax.broadcasted_iota(jnp.int32, (npad, 1), 0)
            m = (rows < n_valid).astype(jnp.float32)
            hm = h * m
            s1 = jnp.sum(hm, axis=0, keepdims=True)
            s2 = jnp.sum(hm * hm, axis=0, keepdims=True)
            mu = s1 / n_valid
            var = s2 / n_valid - mu * mu
            inv = lax.rsqrt(var + _EPS)
            st_ref[0:1, :] = mu
            st_ref[1:2, :] = inv

        mu = st_ref[0:1, :]
        inv = st_ref[1:2, :]
        hb = hp_ref[0, pl.ds(i * bm, bm), :] + hp_ref[1, pl.ds(i * bm, bm), :]
        hn = (hb - mu) * inv
        a = a_ref[0]
        hn = jnp.where(hn > 0, hn, a * hn)
        o_ref[...] = jnp.dot(hn, w_ref[...], preferred_element_type=jnp.float32)

    return pl.pallas_call(
        body,
        grid=(grid,),
        in_specs=[
            pl.BlockSpec((2, npad, _COUT), lambda i: (0, 0, 0)),
            pl.BlockSpec((_COUT, cout), lambda i: (0, 0)),
            pl.BlockSpec(memory_space=pltpu.SMEM),
        ],
        out_specs=pl.BlockSpec((bm, cout), lambda i: (i, 0)),
        out_shape=jax.ShapeDtypeStruct((npad, cout), jnp.float32),
        scratch_shapes=[pltpu.VMEM((2, _COUT), jnp.float32)],
    )(hp, wf, alpha)


def _pack_edges(src, dst, kidx, e_pad, dummy_dst):
    pad = e_pad - src.shape[0]
    src = jnp.concatenate([src, jnp.zeros((pad,), jnp.int32)])
    kidx = jnp.concatenate([kidx, jnp.zeros((pad,), jnp.int32)])
    dst = jnp.concatenate([dst, jnp.full((pad,), dummy_dst, jnp.int32)])
    packed = jnp.stack([src, kidx, dst])
    packed = packed.reshape(3, e_pad // _CHUNK, _CHUNK)
    return packed.transpose(1, 0, 2)


_conv1 = _make_edge_conv_mv(_E1P, _NP1, 3, 4)
_conv2 = _make_edge_conv_mv(_E2P, _NP2, _COUT, 2)
_conv3 = _make_edge_conv_mv(_E3P, _NP3, _COUT, 2)


@jax.jit
def kernel(x, src1, dst1, kidx1, src2, dst2, kidx2, src3, dst3, kidx3,
           W1, W2, W3, W4, a1, a2, a3):
    w1t = W1.reshape(_K * 3, _COUT)
    w2t = W2.reshape(_K * _COUT, _COUT)
    w3t = W3.reshape(_K * _COUT, _COUT)

    e1 = _pack_edges(src1, dst1, kidx1, _E1P, _N1)
    e2 = _pack_edges(src2, dst2, kidx2, _E2P, _N2)
    e3 = _pack_edges(src3, dst3, kidx3, _E3P, _N3)

    x16 = jnp.pad(x, ((0, 0), (0, _COUT - 3)))
    h1p = _conv1(x16, e1, w1t)
    h1n = _norm_only(h1p, a1, _N1)
    h2p = _conv2(h1n, e2, w2t)
    h2n = _norm_only(h2p, a2, _N2)
    h3p = _conv3(h2n, e3, w3t)
    out = _norm_expand(h3p, W4, a3, _N3, _NP3)
    return out[:_N3]


# speedup vs baseline: 1.0119x; 1.0119x over previous
import functools
import jax
import jax.numpy as jnp
from jax import lax
from jax.experimental import pallas as pl
from jax.experimental.pallas import tpu as pltpu
from jax.experimental.pallas import tpu_sc as plsc

_K = 27
_COUT = 16
_EPS = 1e-5
_NC = 2
_NS = 16
_NW = _NC * _NS
_CHUNK = 128
_ZC = 64

_N0, _N1, _N2, _N3 = 100000, 25000, 6250, 1563
_NP1, _NP2, _NP3 = 25600, 7168, 2048
_E1P, _E2P, _E3P = 679936, 172032, 49152


def _make_edge_conv(e_pad, np_out):
    e_per_w = e_pad // _NW
    nch = e_per_w // _CHUNK
    npairs = nch // 2
    rpt = np_out // _NS
    zc = next(d for d in (320, 256, 224, 128, 64, 32, 16) if rpt % d == 0)
    mesh = plsc.VectorSubcoreMesh(core_axis_name="c", subcore_axis_name="s")

    @functools.partial(
        pl.kernel,
        out_type=jax.ShapeDtypeStruct((_NC, np_out, _COUT), jnp.float32),
        mesh=mesh,
        compiler_params=pltpu.CompilerParams(use_tc_tiling_on_sc=False),
        scratch_types=[
            pltpu.VMEM_SHARED((np_out, _COUT), jnp.float32),
            pltpu.VMEM((3, _CHUNK), jnp.int32),
            pltpu.VMEM((3, _CHUNK), jnp.int32),
            pltpu.VMEM((_CHUNK,), jnp.int32),
            pltpu.VMEM((_CHUNK,), jnp.int32),
            pltpu.VMEM((_CHUNK,), jnp.int32),
            pltpu.VMEM((_CHUNK,), jnp.int32),
            pltpu.VMEM((_CHUNK, _COUT), jnp.float32),
            pltpu.VMEM((_CHUNK, _COUT), jnp.float32),
            pltpu.VMEM((zc, _COUT), jnp.float32),
            pltpu.SemaphoreType.DMA,
            pltpu.SemaphoreType.DMA,
            pltpu.SemaphoreType.DMA,
            pltpu.SemaphoreType.DMA,
            pltpu.SemaphoreType.DMA,
            pltpu.SemaphoreType.DMA,
        ],
    )
    def conv(tab_hbm, eidx_hbm, out_hbm,
             acc, ibuf0, ibuf1, gbuf0, gbuf1, dbuf0, dbuf1, rows0, rows1,
             zero_v, isem0, isem1, gsem0, gsem1, ssem0, ssem1):
        cid = lax.axis_index("c")
        sid = lax.axis_index("s")
        wid = sid * _NC + cid

        ibuf = (ibuf0, ibuf1)
        gbuf = (gbuf0, gbuf1)
        dbuf = (dbuf0, dbuf1)
        rows = (rows0, rows1)
        isem = (isem0, isem1)
        gsem = (gsem0, gsem1)
        ssem = (ssem0, ssem1)

        def zfill(r, carry):
            zero_v[r] = jnp.zeros((_COUT,), jnp.float32)
            return carry
        lax.fori_loop(0, zc, zfill, 0)

        for i in range(rpt // zc):
            pltpu.async_copy(zero_v, acc.at[pl.ds(sid * rpt + i * zc, zc)],
                             gsem0)
        for i in range(rpt // zc):
            pltpu.make_async_copy(zero_v, acc.at[pl.ds(sid * rpt, zc)],
                                  gsem0).wait()
        plsc.subcore_barrier()

        chbase = wid * nch

        def fire_idx(g, b):
            return pltpu.async_copy(eidx_hbm.at[chbase + g], ibuf[b], isem[b])

        def wait_idx(b):
            pltpu.make_async_copy(eidx_hbm.at[chbase], ibuf[b], isem[b]).wait()

        def fire_gat(b):
            return pltpu.async_copy(tab_hbm.at[gbuf[b]], rows[b], gsem[b])

        def wait_gat(b):
            pltpu.make_async_copy(tab_hbm.at[gbuf[b]], rows[b], gsem[b]).wait()

        def fire_sct(b):
            return pltpu.async_copy(rows[b], acc.at[dbuf[b]], ssem[b], add=True)

        def wait_sct(b):
            pltpu.make_async_copy(rows[b], acc.at[dbuf[b]], ssem[b]).wait()

        def compute_idx(b):
            for j in range(_CHUNK // 16):
                sl = pl.ds(j * 16, 16)
                s = ibuf[b][0, sl]
                k = ibuf[b][1, sl]
                gbuf[b][sl] = s * _K + k
                dbuf[b][sl] = ibuf[b][2, sl]

        fire_idx(0, 0)

        def pair(p, carry):
            for b in (0, 1):
                g = 2 * p + b
                wait_idx(b)

                @pl.when(p >= 1)
                def _():
                    wait_sct(b)

                compute_idx(b)
                fire_gat(b)
                if b == 0:
                    fire_idx(g + 1, 1)
                else:
                    @pl.when(p < npairs - 1)
                    def _():
                        fire_idx(g + 1, 0)

                if b == 0:
                    @pl.when(p >= 1)
                    def _():
                        wait_gat(1)
                        fire_sct(1)
                else:
                    wait_gat(0)
                    fire_sct(0)
            return carry

        lax.fori_loop(0, npairs, pair, 0)

        wait_gat(1)
        fire_sct(1)
        wait_sct(0)
        wait_sct(1)

        plsc.subcore_barrier()

        for i in range(rpt // zc):
            base = sid * rpt + i * zc
            pltpu.async_copy(acc.at[pl.ds(base, zc)],
                             out_hbm.at[cid, pl.ds(base, zc)], gsem0)
        for i in range(rpt // zc):
            base = sid * rpt
            pltpu.make_async_copy(acc.at[pl.ds(base, zc)],
                                  out_hbm.at[cid, pl.ds(base, zc)],
                                  gsem0).wait()

    return conv


def _make_edge_conv_l1(e_pad, np_out):
    e_per_w = e_pad // _NW
    nch = e_per_w // _CHUNK
    npairs = nch // 2
    rpt = np_out // _NS
    zc = next(d for d in (320, 256, 224, 128, 64, 32, 16) if rpt % d == 0)
    mesh = plsc.VectorSubcoreMesh(core_axis_name="c", subcore_axis_name="s")

    @functools.partial(
        pl.kernel,
        out_type=jax.ShapeDtypeStruct((_NC, np_out, _COUT), jnp.float32),
        mesh=mesh,
        compiler_params=pltpu.CompilerParams(use_tc_tiling_on_sc=False),
        scratch_types=[
            pltpu.VMEM_SHARED((np_out, _COUT), jnp.float32),
            pltpu.VMEM((3, _CHUNK), jnp.int32),
            pltpu.VMEM((3, _CHUNK), jnp.int32),
            pltpu.VMEM((_CHUNK,), jnp.int32),
            pltpu.VMEM((_CHUNK,), jnp.int32),
            pltpu.VMEM((_CHUNK,), jnp.int32),
            pltpu.VMEM((_CHUNK,), jnp.int32),
            pltpu.VMEM((_CHUNK,), jnp.int32),
            pltpu.VMEM((_CHUNK,), jnp.int32),
            pltpu.VMEM((_CHUNK, _COUT), jnp.float32),
            pltpu.VMEM((_CHUNK, _COUT), jnp.float32),
            pltpu.VMEM((_CHUNK, _COUT), jnp.float32),
            pltpu.VMEM((_CHUNK, _COUT), jnp.float32),
            pltpu.VMEM((81, _COUT), jnp.float32),
            pltpu.VMEM((zc, _COUT), jnp.float32),
            pltpu.SemaphoreType.DMA,
            pltpu.SemaphoreType.DMA,
            pltpu.SemaphoreType.DMA,
            pltpu.SemaphoreType.DMA,
            pltpu.SemaphoreType.DMA,
            pltpu.SemaphoreType.DMA,
        ],
    )
    def conv1(tab_hbm, eidx_hbm, w_hbm, out_hbm,
              acc, ibuf0, ibuf1, sbuf0, sbuf1, kbuf0, kbuf1, dbuf0, dbuf1,
              xrows0, xrows1, msgs0, msgs1, w_v, zero_v,
              isem0, isem1, gsem0, gsem1, ssem0, ssem1):
        cid = lax.axis_index("c")
        sid = lax.axis_index("s")
        wid = sid * _NC + cid

        ibuf = (ibuf0, ibuf1)
        sbuf = (sbuf0, sbuf1)
        kbuf = (kbuf0, kbuf1)
        dbuf = (dbuf0, dbuf1)
        xrows = (xrows0, xrows1)
        msgs = (msgs0, msgs1)
        isem = (isem0, isem1)
        gsem = (gsem0, gsem1)
        ssem = (ssem0, ssem1)

        pltpu.sync_copy(w_hbm, w_v)

        def zfill(r, carry):
            zero_v[r] = jnp.zeros((_COUT,), jnp.float32)
            return carry
        lax.fori_loop(0, zc, zfill, 0)

        for i in range(rpt // zc):
            pltpu.async_copy(zero_v, acc.at[pl.ds(sid * rpt + i * zc, zc)],
                             gsem0)
        for i in range(rpt // zc):
            pltpu.make_async_copy(zero_v, acc.at[pl.ds(sid * rpt, zc)],
                                  gsem0).wait()
        plsc.subcore_barrier()

        chbase = wid * nch

        def fire_idx(g, b):
            return pltpu.async_copy(eidx_hbm.at[chbase + g], ibuf[b], isem[b])

        def wait_idx(b):
            pltpu.make_async_copy(eidx_hbm.at[chbase], ibuf[b], isem[b]).wait()

        def fire_gat(b):
            return pltpu.async_copy(tab_hbm.at[sbuf[b]], xrows[b], gsem[b])

        def wait_gat(b):
            pltpu.make_async_copy(tab_hbm.at[sbuf[b]], xrows[b], gsem[b]).wait()

        def fire_sct(b):
            return pltpu.async_copy(msgs[b], acc.at[dbuf[b]], ssem[b], add=True)

        def wait_sct(b):
            pltpu.make_async_copy(msgs[b], acc.at[dbuf[b]], ssem[b]).wait()

        def stage_idx(b):
            for j in range(_CHUNK // 16):
                sl = pl.ds(j * 16, 16)
                sbuf[b][sl] = ibuf[b][0, sl]
                kbuf[b][sl] = ibuf[b][1, sl]
                dbuf[b][sl] = ibuf[b][2, sl]

        def compute_msgs(b):
            @plsc.parallel_loop(0, _CHUNK // 16, unroll=4)
            def grp(j):
                kvec = kbuf[b][pl.ds(j * 16, 16)] * 3
                for l in range(16):
                    wb = kvec[l]
                    xrow = xrows[b][j * 16 + l]
                    m = (xrow[0] * w_v[wb]
                         + xrow[1] * w_v[wb + 1]
                         + xrow[2] * w_v[wb + 2])
                    msgs[b][j * 16 + l] = m

        fire_idx(0, 0)

        def pair(p, carry):
            for b in (0, 1):
                g = 2 * p + b
                wait_idx(b)

                @pl.when(p >= 1)
                def _():
                    wait_sct(b)

                stage_idx(b)
                fire_gat(b)
                if b == 0:
                    fire_idx(g + 1, 1)

                    @pl.when(p >= 1)
                    def _():
                        wait_gat(1)
                        compute_msgs(1)
                        fire_sct(1)
                else:
                    @pl.when(p < npairs - 1)
                    def _():
                        fire_idx(g + 1, 0)
                    wait_gat(0)
                    compute_msgs(0)
                    fire_sct(0)
            return carry

        lax.fori_loop(0, npairs, pair, 0)

        wait_gat(1)
        compute_msgs(1)
        fire_sct(1)
        wait_sct(0)
        wait_sct(1)

        plsc.subcore_barrier()

        for i in range(rpt // zc):
            base = sid * rpt + i * zc
            pltpu.async_copy(acc.at[pl.ds(base, zc)],
                             out_hbm.at[cid, pl.ds(base, zc)], gsem0)
        for i in range(rpt // zc):
            base = sid * rpt
            pltpu.make_async_copy(acc.at[pl.ds(base, zc)],
                                  out_hbm.at[cid, pl.ds(base, zc)],
                                  gsem0).wait()

    return conv1


def _norm_expand(hp, wf, alpha, n_valid, bm):
    npad = hp.shape[1]
    cout = wf.shape[1]
    grid = npad // bm

    def body(hp_ref, w_ref, a_ref, o_ref, st_ref):
        i = pl.program_id(0)

        @pl.when(i == 0)
        def _():
            h = hp_ref[0] + hp_ref[1]
            rows = lax.broadcasted_iota(jnp.int32, (npad, 1), 0)
            m = (rows < n_valid).astype(jnp.float32)
            hm = h * m
            s1 = jnp.sum(hm, axis=0, keepdims=True)
            s2 = jnp.sum(hm * hm, axis=0, keepdims=True)
            mu = s1 / n_valid
            var = s2 / n_valid - mu * mu
            inv = lax.rsqrt(var + _EPS)
            st_ref[0:1, :] = mu
            st_ref[1:2, :] = inv

        mu = st_ref[0:1, :]
        inv = st_ref[1:2, :]
        hb = hp_ref[0, pl.ds(i * bm, bm), :] + hp_ref[1, pl.ds(i * bm, bm), :]
        hn = (hb - mu) * inv
        a = a_ref[0]
        hn = jnp.where(hn > 0, hn, a * hn)
        o_ref[...] = jnp.dot(hn, w_ref[...], preferred_element_type=jnp.float32)

    return pl.pallas_call(
        body,
        grid=(grid,),
        in_specs=[
            pl.BlockSpec((2, npad, _COUT), lambda i: (0, 0, 0)),
            pl.BlockSpec((_COUT, cout), lambda i: (0, 0)),
            pl.BlockSpec(memory_space=pltpu.SMEM),
        ],
        out_specs=pl.BlockSpec((bm, cout), lambda i: (i, 0)),
        out_shape=jax.ShapeDtypeStruct((npad, cout), jnp.float32),
        scratch_shapes=[pltpu.VMEM((2, _COUT), jnp.float32)],
    )(hp, wf, alpha)


def _pack_edges(src, dst, kidx, e_pad, dummy_dst):
    pad = e_pad - src.shape[0]
    src = jnp.concatenate([src, jnp.zeros((pad,), jnp.int32)])
    kidx = jnp.concatenate([kidx, jnp.zeros((pad,), jnp.int32)])
    dst = jnp.concatenate([dst, jnp.full((pad,), dummy_dst, jnp.int32)])
    packed = jnp.stack([src, kidx, dst])
    packed = packed.reshape(3, e_pad // _CHUNK, _CHUNK)
    return packed.transpose(1, 0, 2)


_conv1 = _make_edge_conv_l1(_E1P, _NP1)
_conv2 = _make_edge_conv(_E2P, _NP2)
_conv3 = _make_edge_conv(_E3P, _NP3)


@jax.jit
def kernel(x, src1, dst1, kidx1, src2, dst2, kidx2, src3, dst3, kidx3,
           W1, W2, W3, W4, a1, a2, a3):
    w1t = W1.reshape(_K * 3, _COUT)
    w2f = W2.transpose(1, 0, 2).reshape(_COUT, _K * _COUT)
    w3f = W3.transpose(1, 0, 2).reshape(_COUT, _K * _COUT)

    e1 = _pack_edges(src1, dst1, kidx1, _E1P, _N1)
    e2 = _pack_edges(src2, dst2, kidx2, _E2P, _N2)
    e3 = _pack_edges(src3, dst3, kidx3, _E3P, _N3)

    x16 = jnp.pad(x, ((0, 0), (0, _COUT - 3)))
    h1p = _conv1(x16, e1, w1t)
    y2 = _norm_expand(h1p, w2f, a1, _N1, 1024)
    h2p = _conv2(y2.reshape(-1, _COUT), e2)
    y3 = _norm_expand(h2p, w3f, a2, _N2, 1024)
    h3p = _conv3(y3.reshape(-1, _COUT), e3)
    out = _norm_expand(h3p, W4, a3, _N3, _NP3)
    return out[:_N3]
